# baseline (speedup 1.0000x reference)
import functools

import jax
import jax.numpy as jnp
from jax import lax
from jax.experimental import pallas as pl
from jax.experimental.pallas import tpu as pltpu
from jax.experimental.pallas import tpu_sc as plsc

_N = 50000
_D = 128
_DH = 64
_E = 300000
_NC = 2
_NS = 16
_CH = 128
_NCH = 148
_ES = _CH * _NCH
_EPAD = _ES * _NS
_BR = 12504
_AR = 12544
_TRASH = 12512
_STRIPE = 784
_DSTPAD = 1 << 29

_ROWS_BLK = 2000
_GRID = _N // _ROWS_BLK


def _gelu(x):
    return 0.5 * x * (1.0 + lax.erf(x * 0.7071067811865476))


def _split_out_specs():
    return [
        pl.BlockSpec((_ROWS_BLK, _DH), lambda i: (i, 0)),
        pl.BlockSpec((_ROWS_BLK, _DH), lambda i: (i, 0)),
    ]


def _split_out_shape():
    return [
        jax.ShapeDtypeStruct((_N, _DH), jnp.float32),
        jax.ShapeDtypeStruct((_N, _DH), jnp.float32),
    ]


def _proj_body(x_ref, w_ref, b_ref, lo_ref, hi_ref):
    y = jnp.dot(x_ref[...], w_ref[...], preferred_element_type=jnp.float32)
    y = _gelu(y + b_ref[...])
    lo_ref[...] = y[:, :_DH]
    hi_ref[...] = y[:, _DH:]


def _tc_proj(x, w, b):
    return pl.pallas_call(
        _proj_body,
        grid=(_GRID,),
        in_specs=[
            pl.BlockSpec((_ROWS_BLK, _D), lambda i: (i, 0)),
            pl.BlockSpec((_D, _D), lambda i: (0, 0)),
            pl.BlockSpec((1, _D), lambda i: (0, 0)),
        ],
        out_specs=_split_out_specs(),
        out_shape=_split_out_shape(),
    )(x, w, b.reshape(1, _D))


def _upd_body(sl_ref, sh_ref, c_ref, hl_ref, hh_ref,
              wl_ref, bl_ref, wr_ref, lo_ref, hi_ref):
    rc = 1.0 / jnp.maximum(c_ref[...][:, 0:1], 1.0)
    wl = wl_ref[...]
    wr = wr_ref[...]
    y = jnp.dot(sl_ref[...] * rc, wl[:_DH], preferred_element_type=jnp.float32)
    y = y + jnp.dot(sh_ref[...] * rc, wl[_DH:], preferred_element_type=jnp.float32)
    y = y + bl_ref[...]
    y = y + jnp.dot(hl_ref[...], wr[:_DH], preferred_element_type=jnp.float32)
    y = y + jnp.dot(hh_ref[...], wr[_DH:], preferred_element_type=jnp.float32)
    y = _gelu(y)
    lo_ref[...] = y[:, :_DH]
    hi_ref[...] = y[:, _DH:]


def _tc_upd(s_lo, s_hi, c, h_lo, h_hi, wl, bl, wr):
    return pl.pallas_call(
        _upd_body,
        grid=(_GRID,),
        in_specs=[
            pl.BlockSpec((_ROWS_BLK, _DH), lambda i: (i, 0)),
            pl.BlockSpec((_ROWS_BLK, _DH), lambda i: (i, 0)),
            pl.BlockSpec((_ROWS_BLK, 16), lambda i: (i, 0)),
            pl.BlockSpec((_ROWS_BLK, _DH), lambda i: (i, 0)),
            pl.BlockSpec((_ROWS_BLK, _DH), lambda i: (i, 0)),
            pl.BlockSpec((_D, _D), lambda i: (0, 0)),
            pl.BlockSpec((1, _D), lambda i: (0, 0)),
            pl.BlockSpec((_D, _D), lambda i: (0, 0)),
        ],
        out_specs=_split_out_specs(),
        out_shape=_split_out_shape(),
    )(s_lo, s_hi, c, h_lo, h_hi, wl, bl.reshape(1, _D), wr)


def _outln_body(hl_ref, hh_ref, w_ref, b_ref, g_ref, bb_ref, o_ref):
    w = w_ref[...]
    y = jnp.dot(hl_ref[...], w[:_DH], preferred_element_type=jnp.float32)
    y = y + jnp.dot(hh_ref[...], w[_DH:], preferred_element_type=jnp.float32)
    y = y + b_ref[...]
    mu = jnp.mean(y, axis=-1, keepdims=True)
    v = jnp.mean((y - mu) ** 2, axis=-1, keepdims=True)
    o_ref[...] = (y - mu) * lax.rsqrt(v + 1e-5) * g_ref[...] + bb_ref[...]


def _tc_outln(h_lo, h_hi, w, b, g, bb):
    return pl.pallas_call(
        _outln_body,
        grid=(_GRID,),
        in_specs=[
            pl.BlockSpec((_ROWS_BLK, _DH), lambda i: (i, 0)),
            pl.BlockSpec((_ROWS_BLK, _DH), lambda i: (i, 0)),
            pl.BlockSpec((_D, _D), lambda i: (0, 0)),
            pl.BlockSpec((1, _D), lambda i: (0, 0)),
            pl.BlockSpec((1, _D), lambda i: (0, 0)),
            pl.BlockSpec((1, _D), lambda i: (0, 0)),
        ],
        out_specs=pl.BlockSpec((_ROWS_BLK, _D), lambda i: (i, 0)),
        out_shape=jax.ShapeDtypeStruct((_N, _D), jnp.float32),
    )(h_lo, h_hi, w, b.reshape(1, _D), g.reshape(1, _D), bb.reshape(1, _D))


_MESH = plsc.VectorSubcoreMesh(
    core_axis_name="c", subcore_axis_name="s", num_cores=_NC, num_subcores=_NS
)


@functools.partial(
    pl.kernel,
    out_type=[
        jax.ShapeDtypeStruct((_N, _DH), jnp.float32),
        jax.ShapeDtypeStruct((_N, _DH), jnp.float32),
        jax.ShapeDtypeStruct((_N, 16), jnp.float32),
        jax.ShapeDtypeStruct((_N, _DH), jnp.float32),
        jax.ShapeDtypeStruct((_N, _DH), jnp.float32),
        jax.ShapeDtypeStruct((_N, 16), jnp.float32),
    ],
    mesh=_MESH,
    scratch_types=[
        pltpu.VMEM((_CH,), jnp.int32),
        pltpu.VMEM((_CH,), jnp.int32),
        pltpu.VMEM((_CH,), jnp.int32),
        pltpu.VMEM((_CH, _DH), jnp.float32),
        pltpu.VMEM((_CH, _DH), jnp.float32),
        pltpu.VMEM((_CH, 16), jnp.float32),
        pltpu.VMEM((_CH, 16), jnp.float32),
        pltpu.VMEM_SHARED((_AR, _DH), jnp.float32),
        pltpu.VMEM_SHARED((_AR, 16), jnp.float32),
        pltpu.SemaphoreType.DMA,
    ],
    compiler_params=pltpu.CompilerParams(use_tc_tiling_on_sc=False),
)
def _sc_segsum(hu_l, hu_h, hi_l, hi_h, suc, duc, siu, diu, zpat, zpat16, opat,
               si_l, si_h, cnt_i, su_l, su_h, cnt_u,
               sidx, didx, cidx, rows, zbuf, zbuf16, ones, acc, cacc, sem):
    c = lax.axis_index("c")
    s = lax.axis_index("s")
    pltpu.sync_copy(zpat, zbuf)
    pltpu.sync_copy(zpat16, zbuf16)
    pltpu.sync_copy(opat, ones)

    def run_rel(tables, src_h, dst_h, out_sums, out_cnt):
        for p in range(2):
            lo = (2 * p + c) * _BR
            hi_b = lo + _BR
            for f in range(2):
                table = tables[f]
                out_sum = out_sums[f]
                do_cnt = f == 0
                for k in range(6):
                    pltpu.sync_copy(zbuf, acc.at[pl.ds(s * _STRIPE + k * _CH, _CH)])
                    if do_cnt:
                        pltpu.sync_copy(zbuf16,
                                        cacc.at[pl.ds(s * _STRIPE + k * _CH, _CH)])
                pltpu.sync_copy(zbuf.at[pl.ds(0, 16)],
                                acc.at[pl.ds(s * _STRIPE + 6 * _CH, 16)])
                if do_cnt:
                    pltpu.sync_copy(zbuf16.at[pl.ds(0, 16)],
                                    cacc.at[pl.ds(s * _STRIPE + 6 * _CH, 16)])
                plsc.subcore_barrier()

                def body(g, carry):
                    off = s * _ES + g * _CH
                    pltpu.sync_copy(src_h.at[pl.ds(off, _CH)], sidx)
                    pltpu.sync_copy(dst_h.at[pl.ds(off, _CH)], didx)
                    cp = pltpu.async_copy(table.at[sidx], rows, sem)
                    for t in range(8):
                        dv = didx[pl.ds(16 * t, 16)]
                        m = (dv >= lo) & (dv < hi_b)
                        cidx[pl.ds(16 * t, 16)] = jnp.where(m, dv - lo, _TRASH)
                    cp.wait()
                    pltpu.sync_copy(rows, acc.at[cidx], add=True)
                    if do_cnt:
                        pltpu.sync_copy(ones, cacc.at[cidx], add=True)
                    return carry

                lax.fori_loop(0, _NCH, body, 0)
                plsc.subcore_barrier()

                @pl.when(s < _NS - 1)
                def _():
                    pltpu.sync_copy(acc.at[pl.ds(s * _STRIPE, _STRIPE)],
                                    out_sum.at[pl.ds(lo + s * _STRIPE, _STRIPE)])
                    if do_cnt:
                        pltpu.sync_copy(cacc.at[pl.ds(s * _STRIPE, _STRIPE)],
                                        out_cnt.at[pl.ds(lo + s * _STRIPE, _STRIPE)])

                def _flush_tail(rows_n):
                    pltpu.sync_copy(acc.at[pl.ds(15 * _STRIPE, rows_n)],
                                    out_sum.at[pl.ds(lo + 15 * _STRIPE, rows_n)])
                    if do_cnt:
                        pltpu.sync_copy(cacc.at[pl.ds(15 * _STRIPE, rows_n)],
                                        out_cnt.at[pl.ds(lo + 15 * _STRIPE, rows_n)])

                if p == 0:
                    @pl.when(s == _NS - 1)
                    def _():
                        _flush_tail(_BR - 15 * _STRIPE)
                else:
                    @pl.when((s == _NS - 1) & (c == 0))
                    def _():
                        _flush_tail(_BR - 15 * _STRIPE)

                    @pl.when((s == _NS - 1) & (c == 1))
                    def _():
                        _flush_tail(50000 - 3 * _BR - 15 * _STRIPE)

    run_rel((hu_l, hu_h), suc, duc, (si_l, si_h), cnt_i)
    run_rel((hi_l, hi_h), siu, diu, (su_l, su_h), cnt_u)


def _pad_edges(e):
    pad = _EPAD - _E
    src = jnp.concatenate([e[0], jnp.zeros((pad,), jnp.int32)])
    dst = jnp.concatenate([e[1], jnp.full((pad,), _DSTPAD, jnp.int32)])
    return src, dst


def kernel(x_user, x_item, params, edge_uc, edge_iu):
    p = params
    hu_l, hu_h = _tc_proj(x_user, p["Win_user"], p["bin_user"])
    hi_l, hi_h = _tc_proj(x_item, p["Win_item"], p["bin_item"])
    suc, duc = _pad_edges(edge_uc)
    siu, diu = _pad_edges(edge_iu)
    zpat = jnp.zeros((_CH, _DH), jnp.float32)
    zpat16 = jnp.zeros((_CH, 16), jnp.float32)
    opat = zpat16.at[:, 0].set(1.0)
    for l in range(2):
        si_l, si_h, cnt_i, su_l, su_h, cnt_u = _sc_segsum(
            hu_l, hu_h, hi_l, hi_h, suc, duc, siu, diu, zpat, zpat16, opat)
        ni_l, ni_h = _tc_upd(si_l, si_h, cnt_i, hi_l, hi_h,
                             p[f"Wl_{l}_uc"], p[f"bl_{l}_uc"], p[f"Wr_{l}_uc"])
        hu_l, hu_h = _tc_upd(su_l, su_h, cnt_u, hu_l, hu_h,
                             p[f"Wl_{l}_iu"], p[f"bl_{l}_iu"], p[f"Wr_{l}_iu"])
        hi_l, hi_h = ni_l, ni_h
    out_u = _tc_outln(hu_l, hu_h, p["Wout_user"], p["bout_user"],
                      p["ln_g_user"], p["ln_b_user"])
    out_i = _tc_outln(hi_l, hi_h, p["Wout_item"], p["bout_item"],
                      p["ln_g_item"], p["ln_b_item"])
    return (out_u, out_i)


# speedup vs baseline: 1.1974x; 1.1974x over previous
import functools

import jax
import jax.numpy as jnp
from jax import lax
from jax.experimental import pallas as pl
from jax.experimental.pallas import tpu as pltpu
from jax.experimental.pallas import tpu_sc as plsc

_N = 50000
_D = 128
_DH = 64
_E = 300000
_NC = 2
_NS = 16
_CH = 128
_NCH = 148
_ES = _CH * _NCH
_EPAD = _ES * _NS
_BR = 12504
_AR = 12544
_TRASH = 12512
_STRIPE = 784
_DSTPAD = 1 << 29
_SCALE = 4194304.0
_INV_SCALE = 1.0 / _SCALE

_ROWS_BLK = 2000
_GRID = _N // _ROWS_BLK


def _gelu(x):
    return 0.5 * x * (1.0 + lax.erf(x * 0.7071067811865476))


def _quant(y):
    return (y * _SCALE).astype(jnp.int32)


def _dequant(q):
    return q.astype(jnp.float32) * _INV_SCALE


def _split_out_specs():
    return [
        pl.BlockSpec((_ROWS_BLK, _DH), lambda i: (i, 0)),
        pl.BlockSpec((_ROWS_BLK, _DH), lambda i: (i, 0)),
    ]


def _split_out_shape():
    return [
        jax.ShapeDtypeStruct((_N, _DH), jnp.int32),
        jax.ShapeDtypeStruct((_N, _DH), jnp.int32),
    ]


def _proj_body(x_ref, w_ref, b_ref, lo_ref, hi_ref):
    y = jnp.dot(x_ref[...], w_ref[...], preferred_element_type=jnp.float32)
    q = _quant(_gelu(y + b_ref[...]))
    lo_ref[...] = q[:, :_DH]
    hi_ref[...] = q[:, _DH:]


def _tc_proj(x, w, b):
    return pl.pallas_call(
        _proj_body,
        grid=(_GRID,),
        in_specs=[
            pl.BlockSpec((_ROWS_BLK, _D), lambda i: (i, 0)),
            pl.BlockSpec((_D, _D), lambda i: (0, 0)),
            pl.BlockSpec((1, _D), lambda i: (0, 0)),
        ],
        out_specs=_split_out_specs(),
        out_shape=_split_out_shape(),
    )(x, w, b.reshape(1, _D))


def _upd_body(sl_ref, sh_ref, c_ref, hl_ref, hh_ref,
              wl_ref, bl_ref, wr_ref, lo_ref, hi_ref):
    rc = _INV_SCALE / jnp.maximum(c_ref[...][:, 0:1], 1.0)
    wl = wl_ref[...]
    wr = wr_ref[...]
    ml = sl_ref[...].astype(jnp.float32) * rc
    mh = sh_ref[...].astype(jnp.float32) * rc
    y = jnp.dot(ml, wl[:_DH], preferred_element_type=jnp.float32)
    y = y + jnp.dot(mh, wl[_DH:], preferred_element_type=jnp.float32)
    y = y + bl_ref[...]
    y = y + jnp.dot(_dequant(hl_ref[...]), wr[:_DH],
                    preferred_element_type=jnp.float32)
    y = y + jnp.dot(_dequant(hh_ref[...]), wr[_DH:],
                    preferred_element_type=jnp.float32)
    q = _quant(_gelu(y))
    lo_ref[...] = q[:, :_DH]
    hi_ref[...] = q[:, _DH:]


def _tc_upd(s_lo, s_hi, c, h_lo, h_hi, wl, bl, wr):
    return pl.pallas_call(
        _upd_body,
        grid=(_GRID,),
        in_specs=[
            pl.BlockSpec((_ROWS_BLK, _DH), lambda i: (i, 0)),
            pl.BlockSpec((_ROWS_BLK, _DH), lambda i: (i, 0)),
            pl.BlockSpec((_ROWS_BLK, 16), lambda i: (i, 0)),
            pl.BlockSpec((_ROWS_BLK, _DH), lambda i: (i, 0)),
            pl.BlockSpec((_ROWS_BLK, _DH), lambda i: (i, 0)),
            pl.BlockSpec((_D, _D), lambda i: (0, 0)),
            pl.BlockSpec((1, _D), lambda i: (0, 0)),
            pl.BlockSpec((_D, _D), lambda i: (0, 0)),
        ],
        out_specs=_split_out_specs(),
        out_shape=_split_out_shape(),
    )(s_lo, s_hi, c, h_lo, h_hi, wl, bl.reshape(1, _D), wr)


def _outln_body(hl_ref, hh_ref, w_ref, b_ref, g_ref, bb_ref, o_ref):
    w = w_ref[...]
    y = jnp.dot(_dequant(hl_ref[...]), w[:_DH],
                preferred_element_type=jnp.float32)
    y = y + jnp.dot(_dequant(hh_ref[...]), w[_DH:],
                    preferred_element_type=jnp.float32)
    y = y + b_ref[...]
    mu = jnp.mean(y, axis=-1, keepdims=True)
    v = jnp.mean((y - mu) ** 2, axis=-1, keepdims=True)
    o_ref[...] = (y - mu) * lax.rsqrt(v + 1e-5) * g_ref[...] + bb_ref[...]


def _tc_outln(h_lo, h_hi, w, b, g, bb):
    return pl.pallas_call(
        _outln_body,
        grid=(_GRID,),
        in_specs=[
            pl.BlockSpec((_ROWS_BLK, _DH), lambda i: (i, 0)),
            pl.BlockSpec((_ROWS_BLK, _DH), lambda i: (i, 0)),
            pl.BlockSpec((_D, _D), lambda i: (0, 0)),
            pl.BlockSpec((1, _D), lambda i: (0, 0)),
            pl.BlockSpec((1, _D), lambda i: (0, 0)),
            pl.BlockSpec((1, _D), lambda i: (0, 0)),
        ],
        out_specs=pl.BlockSpec((_ROWS_BLK, _D), lambda i: (i, 0)),
        out_shape=jax.ShapeDtypeStruct((_N, _D), jnp.float32),
    )(h_lo, h_hi, w, b.reshape(1, _D), g.reshape(1, _D), bb.reshape(1, _D))


_MESH = plsc.VectorSubcoreMesh(
    core_axis_name="c", subcore_axis_name="s", num_cores=_NC, num_subcores=_NS
)


@functools.partial(
    pl.kernel,
    out_type=[
        jax.ShapeDtypeStruct((_N, _DH), jnp.int32),
        jax.ShapeDtypeStruct((_N, _DH), jnp.int32),
        jax.ShapeDtypeStruct((_N, 16), jnp.float32),
        jax.ShapeDtypeStruct((_N, _DH), jnp.int32),
        jax.ShapeDtypeStruct((_N, _DH), jnp.int32),
        jax.ShapeDtypeStruct((_N, 16), jnp.float32),
    ],
    mesh=_MESH,
    scratch_types=[
        pltpu.VMEM((_ES,), jnp.int32),
        pltpu.VMEM((_ES,), jnp.int32),
        pltpu.VMEM((_CH,), jnp.int32),
        pltpu.VMEM((_CH,), jnp.int32),
        pltpu.VMEM((_CH, _DH), jnp.int32),
        pltpu.VMEM((_CH, _DH), jnp.int32),
        pltpu.VMEM((_CH, _DH), jnp.int32),
        pltpu.VMEM((_CH, 16), jnp.float32),
        pltpu.VMEM((_CH, 16), jnp.float32),
        pltpu.VMEM_SHARED((_AR, _DH), jnp.int32),
        pltpu.VMEM_SHARED((_AR, 16), jnp.float32),
        pltpu.SemaphoreType.DMA,
        pltpu.SemaphoreType.DMA,
    ],
    compiler_params=pltpu.CompilerParams(use_tc_tiling_on_sc=False),
)
def _sc_segsum(hu_l, hu_h, hi_l, hi_h, suc, duc, siu, diu, zpat, zpat16, opat,
               si_l, si_h, cnt_i, su_l, su_h, cnt_u,
               sall, dall, cidx0, cidx1, rows0, rows1, zbuf, zbuf16, ones,
               acc, cacc, sem0, sem1):
    c = lax.axis_index("c")
    s = lax.axis_index("s")
    pltpu.sync_copy(zpat, zbuf)
    pltpu.sync_copy(zpat16, zbuf16)
    pltpu.sync_copy(opat, ones)

    def run_rel(tables, src_h, dst_h, out_sums, out_cnt):
        pltpu.sync_copy(src_h.at[pl.ds(s * _ES, _ES)], sall)
        pltpu.sync_copy(dst_h.at[pl.ds(s * _ES, _ES)], dall)
        for p in range(2):
            lo = (2 * p + c) * _BR
            hi_b = lo + _BR
            for f in range(2):
                table = tables[f]
                out_sum = out_sums[f]
                do_cnt = f == 0
                for k in range(6):
                    pltpu.sync_copy(zbuf, acc.at[pl.ds(s * _STRIPE + k * _CH, _CH)])
                    if do_cnt:
                        pltpu.sync_copy(zbuf16,
                                        cacc.at[pl.ds(s * _STRIPE + k * _CH, _CH)])
                pltpu.sync_copy(zbuf.at[pl.ds(0, 16)],
                                acc.at[pl.ds(s * _STRIPE + 6 * _CH, 16)])
                if do_cnt:
                    pltpu.sync_copy(zbuf16.at[pl.ds(0, 16)],
                                    cacc.at[pl.ds(s * _STRIPE + 6 * _CH, 16)])
                plsc.subcore_barrier()

                def make_cidx(g, cidx):
                    for t in range(8):
                        dv = dall[pl.ds(g * _CH + 16 * t, 16)]
                        m = (dv >= lo) & (dv < hi_b)
                        cidx[pl.ds(16 * t, 16)] = jnp.where(m, dv - lo, _TRASH)

                def fire(g, rows, sem):
                    return pltpu.async_copy(
                        table.at[sall.at[pl.ds(g * _CH, _CH)]], rows, sem)

                def drain(g, rows, sem, cidx):
                    make_cidx(g, cidx)
                    pltpu.make_async_copy(
                        table.at[sall.at[pl.ds(g * _CH, _CH)]], rows, sem).wait()
                    pltpu.sync_copy(rows, acc.at[cidx], add=True)
                    if do_cnt:
                        pltpu.sync_copy(ones, cacc.at[cidx], add=True)

                fire(0, rows0, sem0)

                def body(j, carry):
                    g0 = 2 * j
                    fire(g0 + 1, rows1, sem1)
                    drain(g0, rows0, sem0, cidx0)

                    @pl.when(j < _NCH // 2 - 1)
                    def _():
                        fire(g0 + 2, rows0, sem0)

                    drain(g0 + 1, rows1, sem1, cidx1)
                    return carry

                lax.fori_loop(0, _NCH // 2, body, 0)
                plsc.subcore_barrier()

                @pl.when(s < _NS - 1)
                def _():
                    pltpu.sync_copy(acc.at[pl.ds(s * _STRIPE, _STRIPE)],
                                    out_sum.at[pl.ds(lo + s * _STRIPE, _STRIPE)])
                    if do_cnt:
                        pltpu.sync_copy(cacc.at[pl.ds(s * _STRIPE, _STRIPE)],
                                        out_cnt.at[pl.ds(lo + s * _STRIPE, _STRIPE)])

                def _flush_tail(rows_n):
                    pltpu.sync_copy(acc.at[pl.ds(15 * _STRIPE, rows_n)],
                                    out_sum.at[pl.ds(lo + 15 * _STRIPE, rows_n)])
                    if do_cnt:
                        pltpu.sync_copy(cacc.at[pl.ds(15 * _STRIPE, rows_n)],
                                        out_cnt.at[pl.ds(lo + 15 * _STRIPE, rows_n)])

                if p == 0:
                    @pl.when(s == _NS - 1)
                    def _():
                        _flush_tail(_BR - 15 * _STRIPE)
                else:
                    @pl.when((s == _NS - 1) & (c == 0))
                    def _():
                        _flush_tail(_BR - 15 * _STRIPE)

                    @pl.when((s == _NS - 1) & (c == 1))
                    def _():
                        _flush_tail(50000 - 3 * _BR - 15 * _STRIPE)

    run_rel((hu_l, hu_h), suc, duc, (si_l, si_h), cnt_i)
    run_rel((hi_l, hi_h), siu, diu, (su_l, su_h), cnt_u)


def _pad_edges(e):
    pad = _EPAD - _E
    src = jnp.concatenate([e[0], jnp.zeros((pad,), jnp.int32)])
    dst = jnp.concatenate([e[1], jnp.full((pad,), _DSTPAD, jnp.int32)])
    return src, dst


def kernel(x_user, x_item, params, edge_uc, edge_iu):
    p = params
    hu_l, hu_h = _tc_proj(x_user, p["Win_user"], p["bin_user"])
    hi_l, hi_h = _tc_proj(x_item, p["Win_item"], p["bin_item"])
    suc, duc = _pad_edges(edge_uc)
    siu, diu = _pad_edges(edge_iu)
    zpat = jnp.zeros((_CH, _DH), jnp.int32)
    zpat16 = jnp.zeros((_CH, 16), jnp.float32)
    opat = zpat16.at[:, 0].set(1.0)
    for l in range(2):
        si_l, si_h, cnt_i, su_l, su_h, cnt_u = _sc_segsum(
            hu_l, hu_h, hi_l, hi_h, suc, duc, siu, diu, zpat, zpat16, opat)
        ni_l, ni_h = _tc_upd(si_l, si_h, cnt_i, hi_l, hi_h,
                             p[f"Wl_{l}_uc"], p[f"bl_{l}_uc"], p[f"Wr_{l}_uc"])
        hu_l, hu_h = _tc_upd(su_l, su_h, cnt_u, hu_l, hu_h,
                             p[f"Wl_{l}_iu"], p[f"bl_{l}_iu"], p[f"Wr_{l}_iu"])
        hi_l, hi_h = ni_l, ni_h
    out_u = _tc_outln(hu_l, hu_h, p["Wout_user"], p["bout_user"],
                      p["ln_g_user"], p["ln_b_user"])
    out_i = _tc_outln(hi_l, hi_h, p["Wout_item"], p["bout_item"],
                      p["ln_g_item"], p["ln_b_item"])
    return (out_u, out_i)
